# baseline (speedup 1.0000x reference)
import functools

import jax
import jax.numpy as jnp
from jax import lax
from jax.experimental import pallas as pl
from jax.experimental.pallas import tpu as pltpu

_K = 100


def _stats_kernel(x_ref, bv_ref, s_ref, npos_ref, m_ref, e_ref, *, hw, nsteps):
    j = pl.program_id(0)
    xb = x_ref[...]
    bm = jnp.max(xb, axis=0)
    bs = jnp.sum(xb, axis=0)
    bnpos = jnp.sum((xb > 0.0).astype(jnp.float32), axis=0)

    @pl.when(j == 0)
    def _init():
        m_ref[...] = bm
        e_ref[...] = jnp.sum(jnp.exp(xb - bm[None]), axis=0)
        s_ref[...] = bs
        npos_ref[...] = bnpos

    @pl.when(j > 0)
    def _update():
        m = m_ref[...]
        new_m = jnp.maximum(m, bm)
        e_ref[...] = e_ref[...] * jnp.exp(m - new_m) + jnp.sum(
            jnp.exp(xb - new_m[None]), axis=0
        )
        m_ref[...] = new_m
        s_ref[...] = s_ref[...] + bs
        npos_ref[...] = npos_ref[...] + bnpos

    @pl.when(j == nsteps - 1)
    def _finish():
        bv_ref[...] = m_ref[...] + jnp.log(e_ref[...]) - s_ref[...] * (1.0 / hw)


def _select_kernel(bv_ref, s_ref, npos_ref, y_ref, loss_ref, np_ref, *, b, c):
    y_row = y_ref[...]
    row = lax.broadcasted_iota(jnp.int32, (c, b), 0)
    is_y = row == y_row

    s = s_ref[...] + 0.0
    s = jnp.where(is_y, -jnp.inf, s)

    bits = pltpu.bitcast(s, jnp.int32)
    key = jnp.where(bits < 0, bits ^ jnp.int32(0x7FFFFFFF), bits)

    lo0 = jnp.full((1, b), jnp.int32(-(2**31)), jnp.int32)
    hi0 = jnp.full((1, b), jnp.int32(2**31 - 1), jnp.int32)

    def body(_, carry):
        lo, hi = carry
        mid = (lo >> 1) + (hi >> 1) + (lo & hi & 1)
        cnt = jnp.sum((key >= mid).astype(jnp.int32), axis=0, keepdims=True)
        ok = cnt >= _K
        return jnp.where(ok, mid, lo), jnp.where(ok, hi, mid)

    lo, _ = lax.fori_loop(0, 32, body, (lo0, hi0))
    thr = lo

    gt = key > thr
    eq = key == thr
    need = (_K - jnp.sum(gt.astype(jnp.int32), axis=0, keepdims=True)).astype(
        jnp.float32
    )
    eq_f = eq.astype(jnp.float32)
    r = lax.broadcasted_iota(jnp.int32, (c, c), 0)
    cc = lax.broadcasted_iota(jnp.int32, (c, c), 1)
    tri = (cc < r).astype(jnp.float32)
    rank = lax.dot_general(
        tri, eq_f, (((1,), (0,)), ((), ())), preferred_element_type=jnp.float32
    )
    sel = gt | (eq & (rank < need))

    bv = bv_ref[...]
    loss_ref[...] = jnp.sum(jnp.where(sel, bv, 0.0), keepdims=True).reshape(1, 1) * (
        1.0 / b
    )

    npos = npos_ref[...]
    col_npos = jnp.sum(npos, axis=0, keepdims=True) - jnp.sum(
        jnp.where(is_y, npos, 0.0), axis=0, keepdims=True
    )
    np_ref[...] = jnp.sum(col_npos.astype(jnp.int32), keepdims=True).reshape(1, 1)


@jax.jit
def kernel(x, y):
    B, C, H, W = x.shape
    HW = H * W
    x_t = jnp.transpose(x, (2, 3, 1, 0)).reshape(HW, C, B)

    HWB = 28
    nsteps = HW // HWB
    bv, s, npos = pl.pallas_call(
        functools.partial(_stats_kernel, hw=HW, nsteps=nsteps),
        grid=(nsteps,),
        in_specs=[pl.BlockSpec((HWB, C, B), lambda j: (j, 0, 0))],
        out_specs=[
            pl.BlockSpec((C, B), lambda j: (0, 0)),
            pl.BlockSpec((C, B), lambda j: (0, 0)),
            pl.BlockSpec((C, B), lambda j: (0, 0)),
        ],
        out_shape=[
            jax.ShapeDtypeStruct((C, B), jnp.float32),
            jax.ShapeDtypeStruct((C, B), jnp.float32),
            jax.ShapeDtypeStruct((C, B), jnp.float32),
        ],
        scratch_shapes=[
            pltpu.VMEM((C, B), jnp.float32),
            pltpu.VMEM((C, B), jnp.float32),
        ],
    )(x_t)

    y2 = y.astype(jnp.int32).reshape(1, B)

    loss, num_posi = pl.pallas_call(
        functools.partial(_select_kernel, b=B, c=C),
        in_specs=[
            pl.BlockSpec((C, B), lambda: (0, 0)),
            pl.BlockSpec((C, B), lambda: (0, 0)),
            pl.BlockSpec((C, B), lambda: (0, 0)),
            pl.BlockSpec((1, B), lambda: (0, 0)),
        ],
        out_specs=[
            pl.BlockSpec((1, 1), lambda: (0, 0)),
            pl.BlockSpec((1, 1), lambda: (0, 0)),
        ],
        out_shape=[
            jax.ShapeDtypeStruct((1, 1), jnp.float32),
            jax.ShapeDtypeStruct((1, 1), jnp.int32),
        ],
    )(bv, s, npos, y2)

    return (loss[0, 0], num_posi[0, 0])
